# Initial kernel scaffold; baseline (speedup 1.0000x reference)
import jax
import jax.numpy as jnp
from jax.experimental import pallas as pl


def kernel(hidden_states_in, edges, edge_features):
    raise NotImplementedError("write your pallas kernel here")


# speedup vs baseline: 3.6389x; 3.6389x over previous
import jax
import jax.numpy as jnp
from jax import lax
from jax.experimental import pallas as pl
from jax.experimental.pallas import tpu as pltpu
from jax.experimental.pallas import tpu_sc as plsc

_E = 320000
_D = 128
_DE = 16
_DOUT = 2 * _D + _DE
_B = 128
_NCHUNKS = _E // _B
_NW = 32


def _mp_body(table_hbm, src_hbm, dst_hbm, ef_hbm, out_hbm,
             sidx_v, didx_v, srows_v, drows_v, ef_v, sem0, sem1):
    wid = lax.axis_index("s") * 2 + lax.axis_index("c")
    per_w = _NCHUNKS // _NW
    rem = _NCHUNKS - per_w * _NW
    start = per_w * wid + jnp.minimum(wid, rem)
    n = per_w + jnp.where(wid < rem, 1, 0)

    def body(i, carry):
        base = (start + i) * _B
        pltpu.sync_copy(src_hbm.at[pl.ds(base, _B)], sidx_v)
        pltpu.sync_copy(dst_hbm.at[pl.ds(base, _B)], didx_v)
        cp0 = pltpu.async_copy(table_hbm.at[sidx_v], srows_v, sem0)
        cp1 = pltpu.async_copy(table_hbm.at[didx_v], drows_v, sem1)
        pltpu.sync_copy(ef_hbm.at[pl.ds(base, _B)], ef_v)
        cp0.wait()
        cp1.wait()
        pltpu.sync_copy(srows_v, out_hbm.at[pl.ds(base, _B), pl.ds(0, _D)])
        pltpu.sync_copy(drows_v, out_hbm.at[pl.ds(base, _B), pl.ds(_D, _D)])
        pltpu.sync_copy(ef_v, out_hbm.at[pl.ds(base, _B), pl.ds(2 * _D, _DE)])
        return carry

    lax.fori_loop(0, n, body, 0)


def kernel(hidden_states_in, edges, edge_features):
    edges32 = edges.astype(jnp.int32)
    src = edges32[:, 0]
    dst = edges32[:, 1]
    mesh = plsc.VectorSubcoreMesh(core_axis_name="c", subcore_axis_name="s")
    f = pl.kernel(
        _mp_body,
        out_type=jax.ShapeDtypeStruct((_E, _DOUT), jnp.float32),
        mesh=mesh,
        scratch_types=[
            pltpu.VMEM((_B,), jnp.int32),
            pltpu.VMEM((_B,), jnp.int32),
            pltpu.VMEM((_B, _D), jnp.float32),
            pltpu.VMEM((_B, _D), jnp.float32),
            pltpu.VMEM((_B, _DE), jnp.float32),
            pltpu.SemaphoreType.DMA,
            pltpu.SemaphoreType.DMA,
        ],
    )
    return f(hidden_states_in, src, dst, edge_features)


# speedup vs baseline: 4.1331x; 1.1358x over previous
import jax
import jax.numpy as jnp
from jax import lax
from jax.experimental import pallas as pl
from jax.experimental.pallas import tpu as pltpu
from jax.experimental.pallas import tpu_sc as plsc

_E = 320000
_D = 128
_DE = 16
_DOUT = 2 * _D + _DE
_B = 80
_NBUF = 4
_NW = 32
_CHUNKS_PER_W = _E // (_B * _NW)
_OUTER = _CHUNKS_PER_W // _NBUF


def _mp_body(table_hbm, src_hbm, dst_hbm, ef_hbm, out_hbm, *scratch):
    sidx = scratch[0:_NBUF]
    didx = scratch[_NBUF:2 * _NBUF]
    srows = scratch[2 * _NBUF:3 * _NBUF]
    drows = scratch[3 * _NBUF:4 * _NBUF]
    efb = scratch[4 * _NBUF:5 * _NBUF]
    semi = scratch[5 * _NBUF:6 * _NBUF]
    semg = scratch[6 * _NBUF:7 * _NBUF]
    semw = scratch[7 * _NBUF:8 * _NBUF]

    wid = lax.axis_index("s") * 2 + lax.axis_index("c")
    start = wid * _CHUNKS_PER_W

    def outer(j, carry):
        def base(b):
            return (start + j * _NBUF + b) * _B

        ins = []
        for b in range(_NBUF):
            c0 = pltpu.async_copy(src_hbm.at[pl.ds(base(b), _B)], sidx[b], semi[b])
            c1 = pltpu.async_copy(dst_hbm.at[pl.ds(base(b), _B)], didx[b], semi[b])
            c2 = pltpu.async_copy(ef_hbm.at[pl.ds(base(b), _B)], efb[b], semi[b])
            ins.append((c0, c1, c2))
        gat = []
        for b in range(_NBUF):
            for c in ins[b]:
                c.wait()
            g0 = pltpu.async_copy(table_hbm.at[sidx[b]], srows[b], semg[b])
            g1 = pltpu.async_copy(table_hbm.at[didx[b]], drows[b], semg[b])
            gat.append((g0, g1))
        outs = []
        for b in range(_NBUF):
            for g in gat[b]:
                g.wait()
            w0 = pltpu.async_copy(srows[b], out_hbm.at[pl.ds(base(b), _B), pl.ds(0, _D)], semw[b])
            w1 = pltpu.async_copy(drows[b], out_hbm.at[pl.ds(base(b), _B), pl.ds(_D, _D)], semw[b])
            w2 = pltpu.async_copy(efb[b], out_hbm.at[pl.ds(base(b), _B), pl.ds(2 * _D, _DE)], semw[b])
            outs.append((w0, w1, w2))
        for b in range(_NBUF):
            for w in outs[b]:
                w.wait()
        return carry

    lax.fori_loop(0, _OUTER, outer, 0)

    for t in range(_OUTER * _NBUF, _CHUNKS_PER_W):
        tb = (start + t) * _B
        pltpu.sync_copy(src_hbm.at[pl.ds(tb, _B)], sidx[0])
        pltpu.sync_copy(dst_hbm.at[pl.ds(tb, _B)], didx[0])
        pltpu.sync_copy(ef_hbm.at[pl.ds(tb, _B)], efb[0])
        pltpu.async_copy(table_hbm.at[sidx[0]], srows[0], semg[0]).wait()
        pltpu.async_copy(table_hbm.at[didx[0]], drows[0], semg[0]).wait()
        pltpu.sync_copy(srows[0], out_hbm.at[pl.ds(tb, _B), pl.ds(0, _D)])
        pltpu.sync_copy(drows[0], out_hbm.at[pl.ds(tb, _B), pl.ds(_D, _D)])
        pltpu.sync_copy(efb[0], out_hbm.at[pl.ds(tb, _B), pl.ds(2 * _D, _DE)])


def kernel(hidden_states_in, edges, edge_features):
    edges32 = edges.astype(jnp.int32)
    src = edges32[:, 0]
    dst = edges32[:, 1]
    mesh = plsc.VectorSubcoreMesh(core_axis_name="c", subcore_axis_name="s")
    scratch = (
        [pltpu.VMEM((_B,), jnp.int32) for _ in range(_NBUF)]
        + [pltpu.VMEM((_B,), jnp.int32) for _ in range(_NBUF)]
        + [pltpu.VMEM((_B, _D), jnp.float32) for _ in range(_NBUF)]
        + [pltpu.VMEM((_B, _D), jnp.float32) for _ in range(_NBUF)]
        + [pltpu.VMEM((_B, _DE), jnp.float32) for _ in range(_NBUF)]
        + [pltpu.SemaphoreType.DMA for _ in range(3 * _NBUF)]
    )
    f = pl.kernel(
        _mp_body,
        out_type=jax.ShapeDtypeStruct((_E, _DOUT), jnp.float32),
        mesh=mesh,
        scratch_types=scratch,
    )
    return f(hidden_states_in, src, dst, edge_features)


# speedup vs baseline: 6.6454x; 1.6078x over previous
import jax
import jax.numpy as jnp
from jax import lax
from jax.experimental import pallas as pl
from jax.experimental.pallas import tpu as pltpu
from jax.experimental.pallas import tpu_sc as plsc

_E = 320000
_D = 128
_DE = 16
_DOUT = 2 * _D + _DE
_B = 80
_NBUF = 4
_NW = 32
_CHUNKS_PER_W = _E // (_B * _NW)
_OUTER = _CHUNKS_PER_W // _NBUF

_EB = 2560


def _gather_body(table_hbm, src_hbm, dst_hbm, a_hbm, b_hbm, *scratch):
    sidx = scratch[0:_NBUF]
    didx = scratch[_NBUF:2 * _NBUF]
    srows = scratch[2 * _NBUF:3 * _NBUF]
    drows = scratch[3 * _NBUF:4 * _NBUF]
    semi = scratch[4 * _NBUF:5 * _NBUF]
    semg = scratch[5 * _NBUF:6 * _NBUF]
    semw = scratch[6 * _NBUF:7 * _NBUF]

    wid = lax.axis_index("s") * 2 + lax.axis_index("c")
    start = wid * _CHUNKS_PER_W

    def outer(j, carry):
        def base(b):
            return (start + j * _NBUF + b) * _B

        ins = []
        for b in range(_NBUF):
            c0 = pltpu.async_copy(src_hbm.at[pl.ds(base(b), _B)], sidx[b], semi[b])
            c1 = pltpu.async_copy(dst_hbm.at[pl.ds(base(b), _B)], didx[b], semi[b])
            ins.append((c0, c1))
        gat = []
        for b in range(_NBUF):
            for c in ins[b]:
                c.wait()
            g0 = pltpu.async_copy(table_hbm.at[sidx[b]], srows[b], semg[b])
            g1 = pltpu.async_copy(table_hbm.at[didx[b]], drows[b], semg[b])
            gat.append((g0, g1))
        outs = []
        for b in range(_NBUF):
            for g in gat[b]:
                g.wait()
            w0 = pltpu.async_copy(srows[b], a_hbm.at[pl.ds(base(b), _B)], semw[b])
            w1 = pltpu.async_copy(drows[b], b_hbm.at[pl.ds(base(b), _B)], semw[b])
            outs.append((w0, w1))
        for b in range(_NBUF):
            for w in outs[b]:
                w.wait()
        return carry

    lax.fori_loop(0, _OUTER, outer, 0)

    for t in range(_OUTER * _NBUF, _CHUNKS_PER_W):
        tb = (start + t) * _B
        pltpu.sync_copy(src_hbm.at[pl.ds(tb, _B)], sidx[0])
        pltpu.sync_copy(dst_hbm.at[pl.ds(tb, _B)], didx[0])
        pltpu.async_copy(table_hbm.at[sidx[0]], srows[0], semg[0]).wait()
        pltpu.async_copy(table_hbm.at[didx[0]], drows[0], semg[0]).wait()
        pltpu.sync_copy(srows[0], a_hbm.at[pl.ds(tb, _B)])
        pltpu.sync_copy(drows[0], b_hbm.at[pl.ds(tb, _B)])


def _sc_gather(hidden_states_in, src, dst):
    mesh = plsc.VectorSubcoreMesh(core_axis_name="c", subcore_axis_name="s")
    scratch = (
        [pltpu.VMEM((_B,), jnp.int32) for _ in range(_NBUF)]
        + [pltpu.VMEM((_B,), jnp.int32) for _ in range(_NBUF)]
        + [pltpu.VMEM((_B, _D), jnp.float32) for _ in range(_NBUF)]
        + [pltpu.VMEM((_B, _D), jnp.float32) for _ in range(_NBUF)]
        + [pltpu.SemaphoreType.DMA for _ in range(3 * _NBUF)]
    )
    f = pl.kernel(
        _gather_body,
        out_type=(
            jax.ShapeDtypeStruct((_E, _D), jnp.float32),
            jax.ShapeDtypeStruct((_E, _D), jnp.float32),
        ),
        mesh=mesh,
        scratch_types=scratch,
    )
    return f(hidden_states_in, src, dst)


def _assemble_body(a_ref, b_ref, eft_ref, out_ref):
    out_ref[pl.ds(0, _D), :] = jnp.transpose(a_ref[...])
    out_ref[pl.ds(_D, _D), :] = jnp.transpose(b_ref[...])
    out_ref[pl.ds(2 * _D, _DE), :] = eft_ref[...]


def _tc_assemble(a, b, eft):
    return pl.pallas_call(
        _assemble_body,
        grid=(_E // _EB,),
        in_specs=[
            pl.BlockSpec((_EB, _D), lambda i: (i, 0)),
            pl.BlockSpec((_EB, _D), lambda i: (i, 0)),
            pl.BlockSpec((_DE, _EB), lambda i: (0, i)),
        ],
        out_specs=pl.BlockSpec((_DOUT, _EB), lambda i: (0, i)),
        out_shape=jax.ShapeDtypeStruct((_DOUT, _E), jnp.float32),
    )(a, b, eft)


def kernel(hidden_states_in, edges, edge_features):
    edges32 = edges.astype(jnp.int32)
    src = edges32[:, 0]
    dst = edges32[:, 1]
    a, b = _sc_gather(hidden_states_in, src, dst)
    tt = _tc_assemble(a, b, edge_features.T)
    return tt.T


# speedup vs baseline: 6.6617x; 1.0025x over previous
import jax
import jax.numpy as jnp
from jax import lax
from jax.experimental import pallas as pl
from jax.experimental.pallas import tpu as pltpu
from jax.experimental.pallas import tpu_sc as plsc

_E = 320000
_D = 128
_DE = 16
_DOUT = 2 * _D + _DE
_B = 80
_NBUF = 4
_NW = 32
_NSLICE = 5
_ES = _E // _NSLICE
_CHUNKS_PER_W = _ES // (_B * _NW)
_OUTER = _CHUNKS_PER_W // _NBUF

_EB = 2560


def _gather_body(slice_idx, table_hbm, src_hbm, dst_hbm, a_hbm, b_hbm, *scratch):
    sidx = scratch[0:_NBUF]
    didx = scratch[_NBUF:2 * _NBUF]
    srows = scratch[2 * _NBUF:3 * _NBUF]
    drows = scratch[3 * _NBUF:4 * _NBUF]
    semi = scratch[4 * _NBUF:5 * _NBUF]
    semg = scratch[5 * _NBUF:6 * _NBUF]
    semw = scratch[6 * _NBUF:7 * _NBUF]

    wid = lax.axis_index("s") * 2 + lax.axis_index("c")
    start = wid * _CHUNKS_PER_W
    src_off = slice_idx * _ES

    def outer(j, carry):
        def base(b):
            return (start + j * _NBUF + b) * _B

        ins = []
        for b in range(_NBUF):
            c0 = pltpu.async_copy(src_hbm.at[pl.ds(src_off + base(b), _B)], sidx[b], semi[b])
            c1 = pltpu.async_copy(dst_hbm.at[pl.ds(src_off + base(b), _B)], didx[b], semi[b])
            ins.append((c0, c1))
        gat = []
        for b in range(_NBUF):
            for c in ins[b]:
                c.wait()
            g0 = pltpu.async_copy(table_hbm.at[sidx[b]], srows[b], semg[b])
            g1 = pltpu.async_copy(table_hbm.at[didx[b]], drows[b], semg[b])
            gat.append((g0, g1))
        outs = []
        for b in range(_NBUF):
            for g in gat[b]:
                g.wait()
            w0 = pltpu.async_copy(srows[b], a_hbm.at[pl.ds(base(b), _B)], semw[b])
            w1 = pltpu.async_copy(drows[b], b_hbm.at[pl.ds(base(b), _B)], semw[b])
            outs.append((w0, w1))
        for b in range(_NBUF):
            for w in outs[b]:
                w.wait()
        return carry

    lax.fori_loop(0, _OUTER, outer, 0)

    for t in range(_OUTER * _NBUF, _CHUNKS_PER_W):
        tb = (start + t) * _B
        pltpu.sync_copy(src_hbm.at[pl.ds(src_off + tb, _B)], sidx[0])
        pltpu.sync_copy(dst_hbm.at[pl.ds(src_off + tb, _B)], didx[0])
        pltpu.async_copy(table_hbm.at[sidx[0]], srows[0], semg[0]).wait()
        pltpu.async_copy(table_hbm.at[didx[0]], drows[0], semg[0]).wait()
        pltpu.sync_copy(srows[0], a_hbm.at[pl.ds(tb, _B)])
        pltpu.sync_copy(drows[0], b_hbm.at[pl.ds(tb, _B)])


def _sc_gather_slice(slice_idx, hidden_states_in, src, dst):
    mesh = plsc.VectorSubcoreMesh(core_axis_name="c", subcore_axis_name="s")
    scratch = (
        [pltpu.VMEM((_B,), jnp.int32) for _ in range(_NBUF)]
        + [pltpu.VMEM((_B,), jnp.int32) for _ in range(_NBUF)]
        + [pltpu.VMEM((_B, _D), jnp.float32) for _ in range(_NBUF)]
        + [pltpu.VMEM((_B, _D), jnp.float32) for _ in range(_NBUF)]
        + [pltpu.SemaphoreType.DMA for _ in range(3 * _NBUF)]
    )
    f = pl.kernel(
        lambda *args: _gather_body(slice_idx, *args),
        out_type=(
            jax.ShapeDtypeStruct((_ES, _D), jnp.float32),
            jax.ShapeDtypeStruct((_ES, _D), jnp.float32),
        ),
        mesh=mesh,
        scratch_types=scratch,
        name=f"sc_gather_s{slice_idx}",
    )
    return f(hidden_states_in, src, dst)


def _assemble_body(a_ref, b_ref, eft_ref, out_ref):
    out_ref[pl.ds(0, _D), :] = jnp.transpose(a_ref[...])
    out_ref[pl.ds(_D, _D), :] = jnp.transpose(b_ref[...])
    out_ref[pl.ds(2 * _D, _DE), :] = eft_ref[...]


def _assemble_body_aliased(tt_ref, a_ref, b_ref, eft_ref, out_ref):
    del tt_ref
    _assemble_body(a_ref, b_ref, eft_ref, out_ref)


def _tc_assemble_slice(slice_idx, tt, a, b, eft):
    nblk = _ES // _EB
    data_specs = [
        pl.BlockSpec((_EB, _D), lambda i: (i, 0)),
        pl.BlockSpec((_EB, _D), lambda i: (i, 0)),
        pl.BlockSpec((_DE, _EB), lambda i, s=slice_idx: (0, s * nblk + i)),
    ]
    common = dict(
        grid=(nblk,),
        out_specs=pl.BlockSpec((_DOUT, _EB), lambda i, s=slice_idx: (0, s * nblk + i)),
        out_shape=jax.ShapeDtypeStruct((_DOUT, _E), jnp.float32),
        name=f"tc_assemble_s{slice_idx}",
    )
    if tt is None:
        return pl.pallas_call(_assemble_body, in_specs=data_specs, **common)(a, b, eft)
    return pl.pallas_call(
        _assemble_body_aliased,
        in_specs=[pl.BlockSpec(memory_space=pl.ANY)] + data_specs,
        input_output_aliases={0: 0},
        **common,
    )(tt, a, b, eft)


def kernel(hidden_states_in, edges, edge_features):
    edges32 = edges.astype(jnp.int32)
    src = edges32[:, 0]
    dst = edges32[:, 1]
    eft = edge_features.T
    parts = [_sc_gather_slice(s, hidden_states_in, src, dst) for s in range(_NSLICE)]
    tt = None
    for s, (a, b) in enumerate(parts):
        tt = _tc_assemble_slice(s, tt, a, b, eft)
    return tt.T


# speedup vs baseline: 6.7002x; 1.0058x over previous
import jax
import jax.numpy as jnp
from jax import lax
from jax.experimental import pallas as pl
from jax.experimental.pallas import tpu as pltpu
from jax.experimental.pallas import tpu_sc as plsc

_E = 320000
_D = 128
_DE = 16
_DOUT = 2 * _D + _DE
_B = 80
_NBUF = 4
_NW = 32
_NSLICE = 5
_ES = _E // _NSLICE
_CHUNKS_PER_W = _ES // (_B * _NW)
_OUTER = _CHUNKS_PER_W // _NBUF

_EB = 2560


def _gather_body(slice_idx, table_hbm, src_hbm, dst_hbm, a_hbm, b_hbm, *scratch):
    sidx = scratch[0:_NBUF]
    didx = scratch[_NBUF:2 * _NBUF]
    srows = scratch[2 * _NBUF:3 * _NBUF]
    drows = scratch[3 * _NBUF:4 * _NBUF]
    semi = scratch[4 * _NBUF:5 * _NBUF]
    semg = scratch[5 * _NBUF:6 * _NBUF]
    semw = scratch[6 * _NBUF:7 * _NBUF]

    wid = lax.axis_index("s") * 2 + lax.axis_index("c")
    start = wid * _CHUNKS_PER_W
    src_off = slice_idx * _ES

    def drain_writes(b):
        pltpu.make_async_copy(srows[b], a_hbm.at[pl.ds(start * _B, _B)], semw[b]).wait()
        pltpu.make_async_copy(drows[b], b_hbm.at[pl.ds(start * _B, _B)], semw[b]).wait()

    def outer(j, carry):
        def base(b):
            return (start + j * _NBUF + b) * _B

        ins = []
        for b in range(_NBUF):
            c0 = pltpu.async_copy(src_hbm.at[pl.ds(src_off + base(b), _B)], sidx[b], semi[b])
            c1 = pltpu.async_copy(dst_hbm.at[pl.ds(src_off + base(b), _B)], didx[b], semi[b])
            ins.append((c0, c1))
        gat = []
        for b in range(_NBUF):
            @pl.when(j > 0)
            def _retire(b=b):
                drain_writes(b)

            for c in ins[b]:
                c.wait()
            g0 = pltpu.async_copy(table_hbm.at[sidx[b]], srows[b], semg[b])
            g1 = pltpu.async_copy(table_hbm.at[didx[b]], drows[b], semg[b])
            gat.append((g0, g1))
        for b in range(_NBUF):
            for g in gat[b]:
                g.wait()
            pltpu.async_copy(srows[b], a_hbm.at[pl.ds(base(b), _B)], semw[b])
            pltpu.async_copy(drows[b], b_hbm.at[pl.ds(base(b), _B)], semw[b])
        return carry

    lax.fori_loop(0, _OUTER, outer, 0)

    for b in range(_NBUF):
        drain_writes(b)

    for t in range(_OUTER * _NBUF, _CHUNKS_PER_W):
        tb = (start + t) * _B
        pltpu.sync_copy(src_hbm.at[pl.ds(src_off + tb, _B)], sidx[0])
        pltpu.sync_copy(dst_hbm.at[pl.ds(src_off + tb, _B)], didx[0])
        pltpu.async_copy(table_hbm.at[sidx[0]], srows[0], semg[0]).wait()
        pltpu.async_copy(table_hbm.at[didx[0]], drows[0], semg[0]).wait()
        pltpu.sync_copy(srows[0], a_hbm.at[pl.ds(tb, _B)])
        pltpu.sync_copy(drows[0], b_hbm.at[pl.ds(tb, _B)])


def _sc_gather_slice(slice_idx, hidden_states_in, src, dst):
    mesh = plsc.VectorSubcoreMesh(core_axis_name="c", subcore_axis_name="s")
    scratch = (
        [pltpu.VMEM((_B,), jnp.int32) for _ in range(_NBUF)]
        + [pltpu.VMEM((_B,), jnp.int32) for _ in range(_NBUF)]
        + [pltpu.VMEM((_B, _D), jnp.float32) for _ in range(_NBUF)]
        + [pltpu.VMEM((_B, _D), jnp.float32) for _ in range(_NBUF)]
        + [pltpu.SemaphoreType.DMA for _ in range(3 * _NBUF)]
    )
    f = pl.kernel(
        lambda *args: _gather_body(slice_idx, *args),
        out_type=(
            jax.ShapeDtypeStruct((_ES, _D), jnp.float32),
            jax.ShapeDtypeStruct((_ES, _D), jnp.float32),
        ),
        mesh=mesh,
        scratch_types=scratch,
        name=f"sc_gather_s{slice_idx}",
    )
    return f(hidden_states_in, src, dst)


def _assemble_body(a_ref, b_ref, eft_ref, out_ref):
    out_ref[pl.ds(0, _D), :] = jnp.transpose(a_ref[...])
    out_ref[pl.ds(_D, _D), :] = jnp.transpose(b_ref[...])
    out_ref[pl.ds(2 * _D, _DE), :] = eft_ref[...]


def _assemble_body_aliased(tt_ref, a_ref, b_ref, eft_ref, out_ref):
    del tt_ref
    _assemble_body(a_ref, b_ref, eft_ref, out_ref)


def _tc_assemble_slice(slice_idx, tt, a, b, eft):
    nblk = _ES // _EB
    data_specs = [
        pl.BlockSpec((_EB, _D), lambda i: (i, 0)),
        pl.BlockSpec((_EB, _D), lambda i: (i, 0)),
        pl.BlockSpec((_DE, _EB), lambda i, s=slice_idx: (0, s * nblk + i)),
    ]
    common = dict(
        grid=(nblk,),
        out_specs=pl.BlockSpec((_DOUT, _EB), lambda i, s=slice_idx: (0, s * nblk + i)),
        out_shape=jax.ShapeDtypeStruct((_DOUT, _E), jnp.float32),
        name=f"tc_assemble_s{slice_idx}",
    )
    if tt is None:
        return pl.pallas_call(_assemble_body, in_specs=data_specs, **common)(a, b, eft)
    return pl.pallas_call(
        _assemble_body_aliased,
        in_specs=[pl.BlockSpec(memory_space=pl.ANY)] + data_specs,
        input_output_aliases={0: 0},
        **common,
    )(tt, a, b, eft)


def kernel(hidden_states_in, edges, edge_features):
    edges32 = edges.astype(jnp.int32)
    src = edges32[:, 0]
    dst = edges32[:, 1]
    eft = edge_features.T
    parts = [_sc_gather_slice(s, hidden_states_in, src, dst) for s in range(_NSLICE)]
    tt = None
    for s, (a, b) in enumerate(parts):
        tt = _tc_assemble_slice(s, tt, a, b, eft)
    return tt.T


# speedup vs baseline: 6.8478x; 1.0220x over previous
import jax
import jax.numpy as jnp
from jax import lax
from jax.experimental import pallas as pl
from jax.experimental.pallas import tpu as pltpu
from jax.experimental.pallas import tpu_sc as plsc

_E = 320000
_D = 128
_DE = 16
_DOUT = 2 * _D + _DE
_B = 80
_NBUF = 4
_NW = 32
_EB = 2560
_UNITS = (4, 25, 32, 32, 28, 4)


def _gather_body(e_off, units, table_hbm, src_hbm, dst_hbm, a_hbm, b_hbm, *scratch):
    sidx = scratch[0:_NBUF]
    didx = scratch[_NBUF:2 * _NBUF]
    srows = scratch[2 * _NBUF:3 * _NBUF]
    drows = scratch[3 * _NBUF:4 * _NBUF]
    semi = scratch[4 * _NBUF:5 * _NBUF]
    semg = scratch[5 * _NBUF:6 * _NBUF]
    semw = scratch[6 * _NBUF:7 * _NBUF]

    wid = lax.axis_index("s") * 2 + lax.axis_index("c")
    nchunks = units
    nouter = nchunks // _NBUF
    start = wid * nchunks
    src_off = e_off

    def drain_writes(b):
        pltpu.make_async_copy(srows[b], a_hbm.at[pl.ds(start * _B, _B)], semw[b]).wait()
        pltpu.make_async_copy(drows[b], b_hbm.at[pl.ds(start * _B, _B)], semw[b]).wait()

    def outer(j, carry):
        def base(b):
            return (start + j * _NBUF + b) * _B

        ins = []
        for b in range(_NBUF):
            c0 = pltpu.async_copy(src_hbm.at[pl.ds(src_off + base(b), _B)], sidx[b], semi[b])
            c1 = pltpu.async_copy(dst_hbm.at[pl.ds(src_off + base(b), _B)], didx[b], semi[b])
            ins.append((c0, c1))
        gat = []
        for b in range(_NBUF):
            @pl.when(j > 0)
            def _retire(b=b):
                drain_writes(b)

            for c in ins[b]:
                c.wait()
            g0 = pltpu.async_copy(table_hbm.at[sidx[b]], srows[b], semg[b])
            g1 = pltpu.async_copy(table_hbm.at[didx[b]], drows[b], semg[b])
            gat.append((g0, g1))
        for b in range(_NBUF):
            for g in gat[b]:
                g.wait()
            pltpu.async_copy(srows[b], a_hbm.at[pl.ds(base(b), _B)], semw[b])
            pltpu.async_copy(drows[b], b_hbm.at[pl.ds(base(b), _B)], semw[b])
        return carry

    lax.fori_loop(0, nouter, outer, 0)

    if nouter > 0:
        for b in range(_NBUF):
            drain_writes(b)

    for t in range(nouter * _NBUF, nchunks):
        tb = (start + t) * _B
        pltpu.sync_copy(src_hbm.at[pl.ds(src_off + tb, _B)], sidx[0])
        pltpu.sync_copy(dst_hbm.at[pl.ds(src_off + tb, _B)], didx[0])
        pltpu.async_copy(table_hbm.at[sidx[0]], srows[0], semg[0]).wait()
        pltpu.async_copy(table_hbm.at[didx[0]], drows[0], semg[0]).wait()
        pltpu.sync_copy(srows[0], a_hbm.at[pl.ds(tb, _B)])
        pltpu.sync_copy(drows[0], b_hbm.at[pl.ds(tb, _B)])


def _sc_gather_slice(slice_idx, e_off, units, hidden_states_in, src, dst):
    mesh = plsc.VectorSubcoreMesh(core_axis_name="c", subcore_axis_name="s")
    scratch = (
        [pltpu.VMEM((_B,), jnp.int32) for _ in range(_NBUF)]
        + [pltpu.VMEM((_B,), jnp.int32) for _ in range(_NBUF)]
        + [pltpu.VMEM((_B, _D), jnp.float32) for _ in range(_NBUF)]
        + [pltpu.VMEM((_B, _D), jnp.float32) for _ in range(_NBUF)]
        + [pltpu.SemaphoreType.DMA for _ in range(3 * _NBUF)]
    )
    es = units * _EB
    f = pl.kernel(
        lambda *args: _gather_body(e_off, units, *args),
        out_type=(
            jax.ShapeDtypeStruct((es, _D), jnp.float32),
            jax.ShapeDtypeStruct((es, _D), jnp.float32),
        ),
        mesh=mesh,
        scratch_types=scratch,
        name=f"sc_gather_s{slice_idx}",
    )
    return f(hidden_states_in, src, dst)


def _assemble_body(a_ref, b_ref, eft_ref, out_ref):
    out_ref[pl.ds(0, _D), :] = jnp.transpose(a_ref[...])
    out_ref[pl.ds(_D, _D), :] = jnp.transpose(b_ref[...])
    out_ref[pl.ds(2 * _D, _DE), :] = eft_ref[...]


def _assemble_body_aliased(tt_ref, a_ref, b_ref, eft_ref, out_ref):
    del tt_ref
    _assemble_body(a_ref, b_ref, eft_ref, out_ref)


def _tc_assemble_slice(slice_idx, u_off, units, tt, a, b, eft):
    nblk = units
    data_specs = [
        pl.BlockSpec((_EB, _D), lambda i: (i, 0)),
        pl.BlockSpec((_EB, _D), lambda i: (i, 0)),
        pl.BlockSpec((_DE, _EB), lambda i, u=u_off: (0, u + i)),
    ]
    common = dict(
        grid=(nblk,),
        out_specs=pl.BlockSpec((_DOUT, _EB), lambda i, u=u_off: (0, u + i)),
        out_shape=jax.ShapeDtypeStruct((_DOUT, _E), jnp.float32),
        name=f"tc_assemble_s{slice_idx}",
    )
    if tt is None:
        return pl.pallas_call(_assemble_body, in_specs=data_specs, **common)(a, b, eft)
    return pl.pallas_call(
        _assemble_body_aliased,
        in_specs=[pl.BlockSpec(memory_space=pl.ANY)] + data_specs,
        input_output_aliases={0: 0},
        **common,
    )(tt, a, b, eft)


def kernel(hidden_states_in, edges, edge_features):
    edges32 = edges.astype(jnp.int32)
    src = edges32[:, 0]
    dst = edges32[:, 1]
    eft = edge_features.T
    offs = []
    u0 = 0
    for u in _UNITS:
        offs.append(u0)
        u0 += u
    parts = [
        _sc_gather_slice(s, offs[s] * _EB, _UNITS[s], hidden_states_in, src, dst)
        for s in range(len(_UNITS))
    ]
    tt = None
    for s, (a, b) in enumerate(parts):
        tt = _tc_assemble_slice(s, offs[s], _UNITS[s], tt, a, b, eft)
    return tt.T
